# baseline (speedup 1.0000x reference)
import functools

import numpy as np
import jax
import jax.numpy as jnp
from jax import lax
from jax.experimental import pallas as pl
from jax.experimental.pallas import tpu as pltpu
import jax.experimental.pallas.tpu_sc as plsc

N_POINTS = 262144
HASH_SIZE = 1 << 19
N_LEVELS = 16
LEVEL_DIM = 2
BASE_RES = 16
RESOLUTION = 500
PER_LEVEL_SCALE = float(np.exp2(np.log2(RESOLUTION / BASE_RES) / (N_LEVELS - 1)))
RES = [int(np.ceil(BASE_RES * (PER_LEVEL_SCALE ** l))) for l in range(N_LEVELS)]
DENSE = [(r + 1) ** 3 <= HASH_SIZE for r in RES]
PRIME1 = np.int32(np.uint32(2654435761).astype(np.int32))
PRIME2 = np.int32(np.uint32(805459861).astype(np.int32))

NC, NS, L = 2, 16, 16
NW = NC * NS
PTS_PER_TILE = N_POINTS // NW
C = 512
NCHUNK = PTS_PER_TILE // C
VPC = C // L
IPR = 128
RPC = C // IPR


def _encode_kernel(xt_hbm, tab_hbm, g_hbm, xv, wv, idxv, lowv, rowsv, gbuf, sem):
    wid = lax.axis_index("s") * NC + lax.axis_index("c")
    tile_base = wid * PTS_PER_TILE
    lane = lax.iota(jnp.int32, L)
    zero16 = jnp.zeros((L,), jnp.int32)
    one16 = jnp.ones((L,), jnp.int32)

    @pl.loop(0, NCHUNK)
    def _chunk(ci):
        pbase = tile_base + ci * C
        pltpu.sync_copy(xt_hbm.at[:, pl.ds(pbase, C)], xv)

        for l in range(N_LEVELS):
            res = RES[l]
            res_f = float(res)
            r1 = res + 1
            row_off = l * HASH_SIZE

            @pl.loop(0, VPC)
            def _pass1(i):
                o = i * L
                pos0 = []
                for d in range(3):
                    xd = xv[d, pl.ds(o, L)]
                    p = (xd + 1.0) * 0.5
                    pos = p * res_f
                    pos0i = pos.astype(jnp.int32)
                    wv[d, pl.ds(o, L)] = pos - pos0i.astype(jnp.float32)
                    pos0.append(pos0i)
                cx0, cy0, cz0 = pos0
                for corner in range(8):
                    cx = cx0 + (corner & 1)
                    cy = cy0 + ((corner >> 1) & 1)
                    cz = cz0 + ((corner >> 2) & 1)
                    if DENSE[l]:
                        idx = cx + cy * r1 + cz * (r1 * r1)
                    else:
                        idx = (cx ^ (cy * PRIME1) ^ (cz * PRIME2)) & (HASH_SIZE - 1)
                    row = idx + row_off
                    idxv[pl.ds(corner * C + o, L)] = lax.shift_right_logical(row, 2)
                    lowv[pl.ds(corner * C + o, L)] = (row & 3) * 2

            pltpu.async_copy(tab_hbm.at[idxv], rowsv, sem).wait()

            @pl.loop(0, VPC)
            def _pass2(i):
                o = i * L
                wx = wv[0, pl.ds(o, L)]
                wy = wv[1, pl.ds(o, L)]
                wz = wv[2, pl.ds(o, L)]
                f0 = jnp.zeros((L,), jnp.float32)
                f1 = jnp.zeros((L,), jnp.float32)
                for corner in range(8):
                    wxc = wx if (corner & 1) else 1.0 - wx
                    wyc = wy if ((corner >> 1) & 1) else 1.0 - wy
                    wzc = wz if ((corner >> 2) & 1) else 1.0 - wz
                    wc = wxc * wyc * wzc
                    r = (corner * C + o) + lane
                    lo = lowv[pl.ds(corner * C + o, L)]
                    v0 = plsc.load_gather(rowsv, [r, lo])
                    v1 = plsc.load_gather(rowsv, [r, lo + one16])
                    f0 = f0 + v0 * wc
                    f1 = f1 + v1 * wc
                gbuf[2 * l, pl.ds(o, L)] = f0
                gbuf[2 * l + 1, pl.ds(o, L)] = f1

        pltpu.sync_copy(gbuf, g_hbm.at[:, pl.ds(pbase, C)])


def _encode(xt, tab_flat):
    mesh = plsc.VectorSubcoreMesh(core_axis_name="c", subcore_axis_name="s")
    grid_dim = N_LEVELS * LEVEL_DIM
    return pl.kernel(
        _encode_kernel,
        out_type=jax.ShapeDtypeStruct((grid_dim, N_POINTS), jnp.float32),
        mesh=mesh,
        scratch_types=[
            pltpu.VMEM((3, C), jnp.float32),
            pltpu.VMEM((3, C), jnp.float32),
            pltpu.VMEM((8 * C,), jnp.int32),
            pltpu.VMEM((8 * C,), jnp.int32),
            pltpu.VMEM((8 * C, 2 * LEVEL_DIM * 2), jnp.float32),
            pltpu.VMEM((grid_dim, C), jnp.float32),
            pltpu.SemaphoreType.DMA,
        ],
        compiler_params=pltpu.CompilerParams(
            needs_layout_passes=False, use_tc_tiling_on_sc=False),
    )(xt, tab_flat)


MLP_BLK = 4096


def _mlp_kernel(g_ref, w1_ref, w2_ref, o_ref):
    g = g_ref[...]
    w1 = w1_ref[...]
    w2 = w2_ref[...]
    h = lax.dot_general(w1, g, (((0,), (0,)), ((), ())),
                        precision=lax.Precision.HIGHEST,
                        preferred_element_type=jnp.float32)
    h = jnp.maximum(h, 0.0)
    o = lax.dot_general(w2, h, (((0,), (0,)), ((), ())),
                        precision=lax.Precision.HIGHEST,
                        preferred_element_type=jnp.float32)
    o_ref[...] = o


def _mlp(g, w1, w2):
    grid = (N_POINTS // MLP_BLK,)
    return pl.pallas_call(
        _mlp_kernel,
        grid=grid,
        in_specs=[
            pl.BlockSpec((32, MLP_BLK), lambda i: (0, i)),
            pl.BlockSpec((32, 32), lambda i: (0, 0)),
            pl.BlockSpec((32, 1), lambda i: (0, 0)),
        ],
        out_specs=pl.BlockSpec((1, MLP_BLK), lambda i: (0, i)),
        out_shape=jax.ShapeDtypeStruct((1, N_POINTS), jnp.float32),
    )(g, w1, w2)


@jax.jit
def kernel(x, tables, W1, W2):
    xt = x.T
    tab_flat = tables.reshape(N_LEVELS * HASH_SIZE // 4, 4 * LEVEL_DIM)
    g = _encode(xt, tab_flat)
    out = _mlp(g, W1, W2)
    return out.reshape(N_POINTS, 1)


# speedup vs baseline: 1.0282x; 1.0282x over previous
import functools

import numpy as np
import jax
import jax.numpy as jnp
from jax import lax
from jax.experimental import pallas as pl
from jax.experimental.pallas import tpu as pltpu
import jax.experimental.pallas.tpu_sc as plsc

N_POINTS = 262144
HASH_SIZE = 1 << 19
N_LEVELS = 16
LEVEL_DIM = 2
BASE_RES = 16
RESOLUTION = 500
PER_LEVEL_SCALE = float(np.exp2(np.log2(RESOLUTION / BASE_RES) / (N_LEVELS - 1)))
RES = [int(np.ceil(BASE_RES * (PER_LEVEL_SCALE ** l))) for l in range(N_LEVELS)]
DENSE = [(r + 1) ** 3 <= HASH_SIZE for r in RES]
PRIME1 = np.int32(np.uint32(2654435761).astype(np.int32))
PRIME2 = np.int32(np.uint32(805459861).astype(np.int32))

NC, NS, L = 2, 16, 16
NW = NC * NS
PTS_PER_TILE = N_POINTS // NW
C = 512
NCHUNK = PTS_PER_TILE // C
VPC = C // L


def _encode_kernel(xt_hbm, tab_hbm, g_hbm, xv, wv, idxv, lowv, rowsv, gbuf,
                   sem0, sem1):
    wid = lax.axis_index("s") * NC + lax.axis_index("c")
    tile_base = wid * PTS_PER_TILE
    lane = lax.iota(jnp.int32, L)
    one16 = jnp.ones((L,), jnp.int32)
    sems = (sem0, sem1)

    def pass1(l, b):
        res_f = float(RES[l])
        r1 = RES[l] + 1
        row_off = l * HASH_SIZE

        @pl.loop(0, VPC)
        def _p1(i):
            o = i * L
            pos0 = []
            for d in range(3):
                p = xv[d, pl.ds(o, L)]
                pos = p * res_f
                pos0i = pos.astype(jnp.int32)
                wv[b, d, pl.ds(o, L)] = pos - pos0i.astype(jnp.float32)
                pos0.append(pos0i)
            cx0, cy0, cz0 = pos0
            for corner in range(8):
                cx = cx0 + (corner & 1)
                cy = cy0 + ((corner >> 1) & 1)
                cz = cz0 + ((corner >> 2) & 1)
                if DENSE[l]:
                    idx = cx + cy * r1 + cz * (r1 * r1)
                else:
                    idx = (cx ^ (cy * PRIME1) ^ (cz * PRIME2)) & (HASH_SIZE - 1)
                row = idx + row_off
                idxv[b, pl.ds(corner * C + o, L)] = lax.shift_right_logical(row, 2)
                lowv[b, pl.ds(corner * C + o, L)] = (row & 3) * 2

        return pltpu.async_copy(tab_hbm.at[idxv.at[b]], rowsv.at[b], sems[b])

    def pass2(l, b):
        b16 = jnp.full((L,), b, jnp.int32)

        @pl.loop(0, VPC)
        def _p2(i):
            o = i * L
            wx = wv[b, 0, pl.ds(o, L)]
            wy = wv[b, 1, pl.ds(o, L)]
            wz = wv[b, 2, pl.ds(o, L)]
            f0 = jnp.zeros((L,), jnp.float32)
            f1 = jnp.zeros((L,), jnp.float32)
            for corner in range(8):
                wxc = wx if (corner & 1) else 1.0 - wx
                wyc = wy if ((corner >> 1) & 1) else 1.0 - wy
                wzc = wz if ((corner >> 2) & 1) else 1.0 - wz
                wc = wxc * wyc * wzc
                r = (corner * C + o) + lane
                lo = lowv[b, pl.ds(corner * C + o, L)]
                v0 = plsc.load_gather(rowsv, [b16, r, lo])
                v1 = plsc.load_gather(rowsv, [b16, r, lo + one16])
                f0 = f0 + v0 * wc
                f1 = f1 + v1 * wc
            gbuf[2 * l, pl.ds(o, L)] = f0
            gbuf[2 * l + 1, pl.ds(o, L)] = f1

    @pl.loop(0, NCHUNK)
    def _chunk(ci):
        pbase = tile_base + ci * C
        pltpu.sync_copy(xt_hbm.at[:, pl.ds(pbase, C)], xv)

        @pl.loop(0, VPC)
        def _prep(i):
            o = i * L
            for d in range(3):
                xv[d, pl.ds(o, L)] = (xv[d, pl.ds(o, L)] + 1.0) * 0.5

        cps = [None, None]
        cps[0] = pass1(0, 0)
        for l in range(N_LEVELS):
            b = l & 1
            if l + 1 < N_LEVELS:
                cps[b ^ 1] = pass1(l + 1, b ^ 1)
            cps[b].wait()
            pass2(l, b)

        pltpu.sync_copy(gbuf, g_hbm.at[:, pl.ds(pbase, C)])


def _encode(xt, tab_blocks):
    mesh = plsc.VectorSubcoreMesh(core_axis_name="c", subcore_axis_name="s")
    grid_dim = N_LEVELS * LEVEL_DIM
    return pl.kernel(
        _encode_kernel,
        out_type=jax.ShapeDtypeStruct((grid_dim, N_POINTS), jnp.float32),
        mesh=mesh,
        scratch_types=[
            pltpu.VMEM((3, C), jnp.float32),
            pltpu.VMEM((2, 3, C), jnp.float32),
            pltpu.VMEM((2, 8 * C), jnp.int32),
            pltpu.VMEM((2, 8 * C), jnp.int32),
            pltpu.VMEM((2, 8 * C, 8), jnp.float32),
            pltpu.VMEM((grid_dim, C), jnp.float32),
            pltpu.SemaphoreType.DMA,
            pltpu.SemaphoreType.DMA,
        ],
        compiler_params=pltpu.CompilerParams(
            needs_layout_passes=False, use_tc_tiling_on_sc=False),
    )(xt, tab_blocks)


MLP_BLK = 4096


def _mlp_kernel(g_ref, w1_ref, w2_ref, o_ref):
    g = g_ref[...]
    w1 = w1_ref[...]
    w2 = w2_ref[...]
    h = lax.dot_general(w1, g, (((0,), (0,)), ((), ())),
                        precision=lax.Precision.HIGHEST,
                        preferred_element_type=jnp.float32)
    h = jnp.maximum(h, 0.0)
    o = lax.dot_general(w2, h, (((0,), (0,)), ((), ())),
                        precision=lax.Precision.HIGHEST,
                        preferred_element_type=jnp.float32)
    o_ref[...] = o


def _mlp(g, w1, w2):
    grid = (N_POINTS // MLP_BLK,)
    return pl.pallas_call(
        _mlp_kernel,
        grid=grid,
        in_specs=[
            pl.BlockSpec((32, MLP_BLK), lambda i: (0, i)),
            pl.BlockSpec((32, 32), lambda i: (0, 0)),
            pl.BlockSpec((32, 1), lambda i: (0, 0)),
        ],
        out_specs=pl.BlockSpec((1, MLP_BLK), lambda i: (0, i)),
        out_shape=jax.ShapeDtypeStruct((1, N_POINTS), jnp.float32),
    )(g, w1, w2)


@jax.jit
def kernel(x, tables, W1, W2):
    xt = x.T
    tab_blocks = tables.reshape(N_LEVELS * HASH_SIZE // 4, 4 * LEVEL_DIM)
    g = _encode(xt, tab_blocks)
    out = _mlp(g, W1, W2)
    return out.reshape(N_POINTS, 1)
